# baseline (speedup 1.0000x reference)
import functools

import jax
import jax.numpy as jnp
from jax import lax
from jax.experimental import pallas as pl
from jax.experimental.pallas import tpu as pltpu
from jax.experimental.pallas import tpu_sc as plsc

_MAG_MASK = 0x7FFFFFFF
_INF_BITS = 0x7F800000

_N = 8 * 32 * 65536
_NW = 32
_PER_W = _N // _NW
_CHUNK = 65536
_NCHUNK = _PER_W // _CHUNK
_LANES = 16


def _sc_partials(xf):
    mesh = plsc.VectorSubcoreMesh(core_axis_name="c", subcore_axis_name="s")

    @functools.partial(
        pl.kernel,
        mesh=mesh,
        out_type=jax.ShapeDtypeStruct((_NW, _LANES), jnp.int32),
        scratch_types=[
            pltpu.VMEM((_CHUNK,), jnp.float32),
            pltpu.VMEM((_LANES,), jnp.int32),
        ],
    )
    def sc_kernel(x_hbm, out_hbm, buf_v, acc_v):
        nc = 2
        wid = lax.axis_index("s") * nc + lax.axis_index("c")
        base = wid * _PER_W

        def chunk_body(c, acc):
            pltpu.sync_copy(x_hbm.at[pl.ds(base + c * _CHUNK, _CHUNK)], buf_v)

            def vec_body(i, a):
                v = buf_v[pl.ds(i * _LANES, _LANES)]
                bits = lax.bitcast_convert_type(v, jnp.int32)
                return jnp.maximum(a, bits & _MAG_MASK)

            return lax.fori_loop(0, _CHUNK // _LANES, vec_body, acc)

        acc = lax.fori_loop(0, _NCHUNK, chunk_body, jnp.zeros((_LANES,), jnp.int32))
        acc_v[...] = acc
        pltpu.sync_copy(acc_v, out_hbm.at[wid])

    return sc_kernel(xf)


def _combine_body(p_ref, out_ref):
    m = jnp.max(p_ref[...])
    out_ref[0, 0] = jnp.where(m <= _INF_BITS, 1, 0).astype(jnp.int32)


@jax.jit
def kernel(x):
    parts = _sc_partials(x.reshape(_N))
    ok = pl.pallas_call(
        _combine_body,
        out_specs=pl.BlockSpec(memory_space=pltpu.SMEM),
        out_shape=jax.ShapeDtypeStruct((1, 1), jnp.int32),
    )(parts)
    return ok[0, 0].astype(jnp.bool_)


# speedup vs baseline: 2.0498x; 2.0498x over previous
import functools

import jax
import jax.numpy as jnp
from jax import lax
from jax.experimental import pallas as pl
from jax.experimental.pallas import tpu as pltpu
from jax.experimental.pallas import tpu_sc as plsc

_MAG_MASK = 0x7FFFFFFF
_INF_BITS = 0x7F800000

_N = 8 * 32 * 65536
_NW = 32
_PER_W = _N // _NW
_CHUNK = 32768
_NCHUNK = _PER_W // _CHUNK
_LANES = 16
_UNROLL = 8


def _sc_partials(xf):
    mesh = plsc.VectorSubcoreMesh(core_axis_name="c", subcore_axis_name="s")

    @functools.partial(
        pl.kernel,
        mesh=mesh,
        out_type=jax.ShapeDtypeStruct((_NW, _LANES), jnp.int32),
        scratch_types=[
            pltpu.VMEM((_CHUNK,), jnp.float32),
            pltpu.VMEM((_CHUNK,), jnp.float32),
            pltpu.VMEM((_LANES,), jnp.int32),
            pltpu.SemaphoreType.DMA,
            pltpu.SemaphoreType.DMA,
        ],
    )
    def sc_kernel(x_hbm, out_hbm, buf0, buf1, acc_v, sem0, sem1):
        wid = lax.axis_index("s") * 2 + lax.axis_index("c")
        base = wid * _PER_W
        bufs = (buf0, buf1)
        sems = (sem0, sem1)

        def start(c):
            return pltpu.async_copy(
                x_hbm.at[pl.ds(base + c * _CHUNK, _CHUNK)], bufs[c % 2], sems[c % 2]
            )

        def reduce_chunk(buf, accs):
            def vec_body(i, a):
                off = i * (_LANES * _UNROLL)
                for k in range(_UNROLL):
                    v = buf[pl.ds(off + k * _LANES, _LANES)]
                    bits = lax.bitcast_convert_type(v, jnp.int32) & _MAG_MASK
                    j = k % 4
                    a = a[:j] + (jnp.maximum(a[j], bits),) + a[j + 1 :]
                return a

            return lax.fori_loop(0, _CHUNK // (_LANES * _UNROLL), vec_body, accs)

        accs = tuple(jnp.zeros((_LANES,), jnp.int32) for _ in range(4))
        pending = start(0)
        for c in range(_NCHUNK):
            nxt = start(c + 1) if c + 1 < _NCHUNK else None
            pending.wait()
            accs = reduce_chunk(bufs[c % 2], accs)
            pending = nxt

        acc = jnp.maximum(
            jnp.maximum(accs[0], accs[1]), jnp.maximum(accs[2], accs[3])
        )
        acc_v[...] = acc
        pltpu.sync_copy(acc_v, out_hbm.at[wid])

    return sc_kernel(xf)


def _combine_body(p_ref, out_ref):
    m = jnp.max(p_ref[...])
    out_ref[0, 0] = jnp.where(m <= _INF_BITS, 1, 0).astype(jnp.int32)


@jax.jit
def kernel(x):
    parts = _sc_partials(x.reshape(_N))
    ok = pl.pallas_call(
        _combine_body,
        out_specs=pl.BlockSpec(memory_space=pltpu.SMEM),
        out_shape=jax.ShapeDtypeStruct((1, 1), jnp.int32),
    )(parts)
    return ok[0, 0].astype(jnp.bool_)


# speedup vs baseline: 2.2476x; 1.0965x over previous
import functools

import jax
import jax.numpy as jnp
from jax import lax
from jax.experimental import pallas as pl
from jax.experimental.pallas import tpu as pltpu
from jax.experimental.pallas import tpu_sc as plsc

_MAG_MASK = 0x7FFFFFFF
_INF_BITS = 0x7F800000

_N = 8 * 32 * 65536
_TC_SLABS = 7
_SC_BASE = _TC_SLABS * 32 * 65536
_NW = 32
_PER_W = (_N - _SC_BASE) // _NW
_CHUNK = 32768
_NCHUNK = _PER_W // _CHUNK
_LANES = 16
_UNROLL = 8


def _sc_partials(xf):
    mesh = plsc.VectorSubcoreMesh(core_axis_name="c", subcore_axis_name="s")

    @functools.partial(
        pl.kernel,
        mesh=mesh,
        out_type=jax.ShapeDtypeStruct((_NW, _LANES), jnp.int32),
        scratch_types=[
            pltpu.VMEM((_CHUNK,), jnp.float32),
            pltpu.VMEM((_CHUNK,), jnp.float32),
            pltpu.VMEM((_LANES,), jnp.int32),
            pltpu.SemaphoreType.DMA,
            pltpu.SemaphoreType.DMA,
        ],
    )
    def sc_kernel(x_hbm, out_hbm, buf0, buf1, acc_v, sem0, sem1):
        wid = lax.axis_index("s") * 2 + lax.axis_index("c")
        base = _SC_BASE + wid * _PER_W
        bufs = (buf0, buf1)
        sems = (sem0, sem1)

        def start(c):
            return pltpu.async_copy(
                x_hbm.at[pl.ds(base + c * _CHUNK, _CHUNK)], bufs[c % 2], sems[c % 2]
            )

        def reduce_chunk(buf, accs):
            def vec_body(i, a):
                off = i * (_LANES * _UNROLL)
                for k in range(_UNROLL):
                    v = buf[pl.ds(off + k * _LANES, _LANES)]
                    bits = lax.bitcast_convert_type(v, jnp.int32) & _MAG_MASK
                    j = k % 4
                    a = a[:j] + (jnp.maximum(a[j], bits),) + a[j + 1 :]
                return a

            return lax.fori_loop(0, _CHUNK // (_LANES * _UNROLL), vec_body, accs)

        accs = tuple(jnp.zeros((_LANES,), jnp.int32) for _ in range(4))
        pending = start(0)
        for c in range(_NCHUNK):
            nxt = start(c + 1) if c + 1 < _NCHUNK else None
            pending.wait()
            accs = reduce_chunk(bufs[c % 2], accs)
            pending = nxt

        acc = jnp.maximum(
            jnp.maximum(accs[0], accs[1]), jnp.maximum(accs[2], accs[3])
        )
        acc_v[...] = acc
        pltpu.sync_copy(acc_v, out_hbm.at[wid])

    return sc_kernel(xf)


def _tc_body(x_ref, out_ref, acc_ref):
    i = pl.program_id(0)
    bits = lax.bitcast_convert_type(x_ref[...], jnp.int32)
    m = jnp.max(bits & _MAG_MASK)

    @pl.when(i == 0)
    def _init():
        acc_ref[0] = m

    @pl.when(i > 0)
    def _acc():
        acc_ref[0] = jnp.maximum(acc_ref[0], m)

    @pl.when(i == _TC_SLABS - 1)
    def _finalize():
        out_ref[0, 0] = acc_ref[0]


def _combine_body(sc_ref, tc_ref, out_ref):
    m = jnp.maximum(jnp.max(sc_ref[...]), tc_ref[0, 0])
    out_ref[0, 0] = jnp.where(m <= _INF_BITS, 1, 0).astype(jnp.int32)


@jax.jit
def kernel(x):
    sc_parts = _sc_partials(x.reshape(_N))
    tc_part = pl.pallas_call(
        _tc_body,
        grid=(_TC_SLABS,),
        in_specs=[pl.BlockSpec((1, 32, 65536), lambda i: (i, 0, 0))],
        out_specs=pl.BlockSpec(
            block_shape=(1, 1),
            index_map=lambda i: (0, 0),
            memory_space=pltpu.SMEM,
        ),
        out_shape=jax.ShapeDtypeStruct((1, 1), jnp.int32),
        scratch_shapes=[pltpu.SMEM((1,), jnp.int32)],
    )(x)
    ok = pl.pallas_call(
        _combine_body,
        in_specs=[
            pl.BlockSpec(memory_space=pltpu.VMEM),
            pl.BlockSpec(memory_space=pltpu.SMEM),
        ],
        out_specs=pl.BlockSpec(memory_space=pltpu.SMEM),
        out_shape=jax.ShapeDtypeStruct((1, 1), jnp.int32),
    )(sc_parts, tc_part)
    return ok[0, 0].astype(jnp.bool_)


# speedup vs baseline: 8.4273x; 3.7494x over previous
import jax
import jax.numpy as jnp
from jax.experimental import pallas as pl
from jax.experimental.pallas import tpu as pltpu

_GRID = 8
_BLK_C = 65536
_MAG_MASK = 0x7FFFFFFF
_INF_BITS = 0x7F800000


def _nan_free_body(x_ref, out_ref, acc_ref):
    i = pl.program_id(0)
    bits = jax.lax.bitcast_convert_type(x_ref[...], jnp.int32)
    m = jnp.max(bits & _MAG_MASK)

    @pl.when(i == 0)
    def _init():
        acc_ref[0] = m

    @pl.when(i > 0)
    def _acc():
        acc_ref[0] = jnp.maximum(acc_ref[0], m)

    @pl.when(i == _GRID - 1)
    def _finalize():
        out_ref[0, 0] = jnp.where(acc_ref[0] <= _INF_BITS, 1, 0).astype(jnp.int32)


@jax.jit
def kernel(x):
    ok = pl.pallas_call(
        _nan_free_body,
        grid=(_GRID,),
        in_specs=[pl.BlockSpec((1, 32, _BLK_C), lambda i: (i, 0, 0))],
        out_specs=pl.BlockSpec(
            block_shape=(1, 1),
            index_map=lambda i: (0, 0),
            memory_space=pltpu.SMEM,
        ),
        out_shape=jax.ShapeDtypeStruct((1, 1), jnp.int32),
        scratch_shapes=[pltpu.SMEM((1,), jnp.int32)],
    )(x)
    return ok[0, 0].astype(jnp.bool_)
